# baseline (speedup 1.0000x reference)
import jax
import jax.numpy as jnp
from jax.experimental import pallas as pl
from jax.experimental.pallas import tpu as pltpu

_DX = 768
_HW = 49
_F = 4096
_M = 16
_FB = 512
_RB = 8192


def _pool_body(x_ref, o_ref):
    ones = jnp.ones((_HW, 1), jnp.float32)
    s = jnp.dot(x_ref[...], ones, preferred_element_type=jnp.float32)
    o_ref[...] = s * (1.0 / _HW)


def _gate_body(p_ref, d_ref, w_ref, b_ref, wg_ref,
               gates_ref, probs_ref, idx_ref, acc_ref):
    f = pl.program_id(0)
    nf = pl.num_programs(0)
    h = jnp.dot(p_ref[...], w_ref[0:_DX, :], preferred_element_type=jnp.float32)
    h = h + jnp.dot(d_ref[...], w_ref[_DX:, :], preferred_element_type=jnp.float32)
    h = h + b_ref[...]
    g = 0.5 * h * (1.0 + jax.lax.erf(h * (2.0 ** -0.5)))
    part = jnp.dot(g, wg_ref[...], preferred_element_type=jnp.float32)

    @pl.when(f == 0)
    def _():
        acc_ref[...] = part

    @pl.when(f != 0)
    def _():
        acc_ref[...] = acc_ref[...] + part

    @pl.when(f == nf - 1)
    def _():
        logits = acc_ref[...]
        rows = logits.shape[0]
        iota = jax.lax.broadcasted_iota(jnp.int32, (rows, _M), 1)
        m1 = jnp.max(logits, axis=1, keepdims=True)
        i1 = jnp.min(jnp.where(logits == m1, iota, _M), axis=1, keepdims=True)
        masked = jnp.where(iota == i1, -jnp.inf, logits)
        m2 = jnp.max(masked, axis=1, keepdims=True)
        i2 = jnp.min(jnp.where(masked == m2, iota, _M), axis=1, keepdims=True)
        e = jnp.exp(logits - m1)
        probs_ref[...] = e / jnp.sum(e, axis=1, keepdims=True)
        ev = jnp.exp(m2 - m1)
        g1 = 1.0 / (1.0 + ev)
        g2 = ev / (1.0 + ev)
        gates_ref[...] = (jnp.where(iota == i1, g1, 0.0)
                          + jnp.where(iota == i2, g2, 0.0))
        idx_ref[...] = jnp.concatenate([i1, i2], axis=1)


def kernel(x, Degraded_feature, W_fusion, b_fusion, w_gate):
    B = x.shape[0]
    xf = x.reshape(B * _DX, _HW)
    pooled = pl.pallas_call(
        _pool_body,
        grid=(B * _DX // _RB,),
        in_specs=[pl.BlockSpec((_RB, _HW), lambda i: (i, 0))],
        out_specs=pl.BlockSpec((_RB, 1), lambda i: (i, 0)),
        out_shape=jax.ShapeDtypeStruct((B * _DX, 1), jnp.float32),
    )(xf)
    pooled = pooled.reshape(B, _DX)
    b2 = b_fusion.reshape(1, _F)
    dt = Degraded_feature.shape[1]
    gates, probs, idx = pl.pallas_call(
        _gate_body,
        grid=(_F // _FB,),
        in_specs=[
            pl.BlockSpec((B, _DX), lambda f: (0, 0)),
            pl.BlockSpec((B, dt), lambda f: (0, 0)),
            pl.BlockSpec((_DX + dt, _FB), lambda f: (0, f)),
            pl.BlockSpec((1, _FB), lambda f: (0, f)),
            pl.BlockSpec((_FB, _M), lambda f: (f, 0)),
        ],
        out_specs=[
            pl.BlockSpec((B, _M), lambda f: (0, 0)),
            pl.BlockSpec((B, _M), lambda f: (0, 0)),
            pl.BlockSpec((B, 2), lambda f: (0, 0)),
        ],
        out_shape=[
            jax.ShapeDtypeStruct((B, _M), jnp.float32),
            jax.ShapeDtypeStruct((B, _M), jnp.float32),
            jax.ShapeDtypeStruct((B, 2), jnp.int32),
        ],
        scratch_shapes=[pltpu.VMEM((B, _M), jnp.float32)],
    )(pooled, Degraded_feature, W_fusion, b2, w_gate)
    moe_loss = jnp.zeros((), jnp.float32)
    return (gates, moe_loss, probs, idx)
